# baseline (speedup 1.0000x reference)
import functools

import jax
import jax.numpy as jnp
from jax import lax
from jax.experimental import pallas as pl
from jax.experimental.pallas import tpu as pltpu
from jax.experimental.pallas import tpu_sc as plsc

E = 8
TOP_K = 2
D_MODEL = 1024
D_FF = 4096
T = 2048
BM = 256
NT = 24
NTOT = NT * BM
BF = 1024
NF = D_FF // BF
EL = 128


def _router_body(x_ref, rw_ref, dest_ref, gates_ref, te_ref):
    i32 = jnp.int32
    x = x_ref[...]
    rw = rw_ref[...]
    logits = jnp.dot(x, rw, preferred_element_type=jnp.float32)
    eidx = lax.broadcasted_iota(i32, (T, EL), 1)
    logits = jnp.where(eidx < E, logits, -1e30)

    m1 = jnp.max(logits, axis=1, keepdims=True)
    i1 = jnp.min(jnp.where(logits == m1, eidx, EL), axis=1, keepdims=True)
    masked = jnp.where(eidx == i1, -1e30, logits)
    m2 = jnp.max(masked, axis=1, keepdims=True)
    i2 = jnp.min(jnp.where(masked == m2, eidx, EL), axis=1, keepdims=True)
    g1 = 1.0 / (1.0 + jnp.exp(m2 - m1))
    g2 = 1.0 - g1

    c1 = (eidx == i1).astype(i32)
    c2 = (eidx == i2).astype(i32)
    cnt = c1 + c2
    inc = cnt
    sh = 1
    while sh < T:
        inc = inc + jnp.concatenate(
            [jnp.zeros((sh, EL), i32), inc[:-sh]], axis=0)
        sh *= 2
    exc = inc - cnt
    counts = inc[T - 1:T, :]
    padded = ((counts + (BM - 1)) // BM) * BM
    start = jnp.zeros((1, EL), i32)
    for k in range(1, E):
        start = start + jnp.concatenate(
            [jnp.zeros((1, k), i32), padded[:, :-k]], axis=1)

    d1 = jnp.sum(c1 * (start + exc), axis=1, keepdims=True)
    d2 = jnp.sum(c2 * (start + exc), axis=1, keepdims=True)
    dest_ref[...] = jnp.concatenate([d1, d2], axis=1)
    gates_ref[...] = jnp.concatenate([g1, g2], axis=1)

    er = lax.broadcasted_iota(i32, (E, E), 0)
    ec = lax.broadcasted_iota(i32, (E, E), 1)
    tf = er * E + ec
    ec1 = lax.broadcasted_iota(i32, (1, EL), 1)
    te = jnp.full((E, E), -1, i32)
    for e in range(E):
        s_e = jnp.sum(jnp.where(ec1 == e, start, 0), axis=1, keepdims=True)
        te = te + (tf * BM >= s_e).astype(i32)
    te_ref[...] = jnp.clip(te, 0, E - 1)


def _router(x, router_w):
    rw = jnp.pad(router_w, ((0, 0), (0, EL - E)))
    return pl.pallas_call(
        _router_body,
        out_shape=[
            jax.ShapeDtypeStruct((T, 2), jnp.int32),
            jax.ShapeDtypeStruct((T, 2), jnp.float32),
            jax.ShapeDtypeStruct((E, E), jnp.int32),
        ],
    )(x, rw)


def _dispatch_body(d0_hbm, d1_hbm, g0_hbm, g1_hbm, x_hbm, xs_hbm, rg_hbm,
                   d_v, g_v, rt_v, rg_v, rows_v, gsems, wsems):
    c = lax.axis_index("c")
    s = lax.axis_index("s")
    w = c * 16 + s
    rows_per = NTOT // 32
    chunk = rows_per // 2
    lo = w * rows_per

    pltpu.sync_copy(d0_hbm, d_v.at[0])
    pltpu.sync_copy(d1_hbm, d_v.at[1])
    pltpu.sync_copy(g0_hbm, g_v.at[0])
    pltpu.sync_copy(g1_hbm, g_v.at[1])

    def zero(i, carry):
        rt_v[pl.ds(i * 16, 16)] = jnp.zeros((16,), jnp.int32)
        rg_v[pl.ds(i * 16, 16)] = jnp.zeros((16,), jnp.float32)
        return carry
    lax.fori_loop(0, rows_per // 16, zero, 0)

    for k in range(2):
        def scat(i, carry):
            sl = pl.ds(i * 16, 16)
            idx = d_v[k, sl]
            local = idx - lo
            m = (idx >= lo) & (local < rows_per)
            toks = lax.iota(jnp.int32, 16) + i * 16
            plsc.store_scatter(rt_v, [local], toks, mask=m)
            plsc.store_scatter(rg_v, [local], g_v[k, sl], mask=m)
            return carry
        lax.fori_loop(0, T // 16, scat, 0)

    pltpu.sync_copy(rg_v, rg_hbm.at[pl.ds(lo, rows_per)])

    nchunk = 6
    cr = rows_per // nchunk

    def gather(k):
        idx_slice = rt_v.at[pl.ds(k * cr, cr)]
        return pltpu.async_copy(x_hbm.at[idx_slice], rows_v.at[k % 3],
                                gsems.at[k % 3])

    def write(k):
        return pltpu.async_copy(rows_v.at[k % 3],
                                xs_hbm.at[pl.ds(lo + k * cr, cr)],
                                wsems.at[k % 3])

    gcp = {k: gather(k) for k in range(3)}
    wcp = {}
    for k in range(nchunk):
        gcp[k].wait()
        wcp[k] = write(k)
        if k + 3 < nchunk:
            wcp[k].wait()
            gcp[k + 3] = gather(k + 3)
    for k in range(nchunk - 3, nchunk):
        wcp[k].wait()


def _dispatch(d0, d1, g0, g1, x):
    mesh = plsc.VectorSubcoreMesh(core_axis_name="c", subcore_axis_name="s")
    rows_per = NTOT // 32
    chunk = rows_per // 2
    return pl.kernel(
        _dispatch_body,
        mesh=mesh,
        compiler_params=pltpu.CompilerParams(needs_layout_passes=False),
        out_type=[
            jax.ShapeDtypeStruct((NTOT, D_MODEL), jnp.float32),
            jax.ShapeDtypeStruct((NTOT,), jnp.float32),
        ],
        scratch_types=[
            pltpu.VMEM((2, T), jnp.int32),
            pltpu.VMEM((2, T), jnp.float32),
            pltpu.VMEM((rows_per,), jnp.int32),
            pltpu.VMEM((rows_per,), jnp.float32),
            pltpu.VMEM((3, rows_per // 6, D_MODEL), jnp.float32),
            pltpu.SemaphoreType.DMA((3,)),
            pltpu.SemaphoreType.DMA((3,)),
        ],
    )(d0, d1, g0, g1, x)


def _gemm_body(te_ref, acc_ref, xs_ref, w1_ref, w3_ref, w2_ref, rg_ref,
               out_ref):
    j = pl.program_id(0)
    x = xs_ref[...]
    w1 = w1_ref[0]
    w3 = w3_ref[0]
    w2 = w2_ref[0]
    a = jnp.dot(x, w1, preferred_element_type=jnp.float32)
    b = jnp.dot(x, w3, preferred_element_type=jnp.float32)
    h = (a / (1.0 + jnp.exp(-a))) * b
    contrib = jnp.dot(h, w2, preferred_element_type=jnp.float32)

    @pl.when(j == 0)
    def _():
        out_ref[...] = contrib

    @pl.when((j > 0) & (j < NF - 1))
    def _():
        out_ref[...] = acc_ref[...] + contrib

    @pl.when(j == NF - 1)
    def _():
        out_ref[...] = (acc_ref[...] + contrib) * rg_ref[0]


def _gemm(te, xs, W1, W3, W2, rg):
    grid_spec = pltpu.PrefetchScalarGridSpec(
        num_scalar_prefetch=1,
        grid=(NF, NT),
        in_specs=[
            pl.BlockSpec((BM, D_MODEL), lambda j, i, te: (i, 0)),
            pl.BlockSpec((BM, D_MODEL), lambda j, i, te: (i, 0)),
            pl.BlockSpec((1, D_MODEL, BF), lambda j, i, te: (te[i], 0, j)),
            pl.BlockSpec((1, D_MODEL, BF), lambda j, i, te: (te[i], 0, j)),
            pl.BlockSpec((1, BF, D_MODEL), lambda j, i, te: (te[i], j, 0)),
            pl.BlockSpec((1, BM, 1), lambda j, i, te: (i, 0, 0)),
        ],
        out_specs=pl.BlockSpec((BM, D_MODEL), lambda j, i, te: (i, 0)),
    )
    acc0 = jnp.zeros((NTOT, D_MODEL), jnp.float32)
    return pl.pallas_call(
        _gemm_body,
        grid_spec=grid_spec,
        out_shape=jax.ShapeDtypeStruct((NTOT, D_MODEL), jnp.float32),
        input_output_aliases={1: 0},
        compiler_params=pltpu.CompilerParams(
            dimension_semantics=("arbitrary", "arbitrary")),
    )(te, acc0, xs, W1, W3, W2, rg.reshape(NT, BM, 1))


def _combine_body(or_hbm, d0_hbm, d1_hbm, out_hbm,
                  idx_v, r0_v, r1_v, sem0, sem1):
    c = lax.axis_index("c")
    s = lax.axis_index("s")
    wid = c * 16 + s
    toks_per = T // 32
    chunk = toks_per // 2
    base = wid * toks_per
    for k in range(2):
        tb = base + k * chunk
        pltpu.sync_copy(d0_hbm.at[pl.ds(tb, chunk)], idx_v.at[0])
        pltpu.sync_copy(d1_hbm.at[pl.ds(tb, chunk)], idx_v.at[1])
        cp0 = pltpu.async_copy(or_hbm.at[idx_v.at[0]], r0_v, sem0)
        cp1 = pltpu.async_copy(or_hbm.at[idx_v.at[1]], r1_v, sem1)
        cp0.wait()
        cp1.wait()

        def row(r, carry):
            for cc in range(D_MODEL // 16):
                sl = pl.ds(cc * 16, 16)
                r0_v[r, sl] = r0_v[r, sl] + r1_v[r, sl]
            return carry
        lax.fori_loop(0, chunk, row, 0)
        pltpu.sync_copy(r0_v, out_hbm.at[pl.ds(tb, chunk)])


def _combine(out_rows, d0, d1):
    mesh = plsc.VectorSubcoreMesh(core_axis_name="c", subcore_axis_name="s")
    chunk = T // 64
    return pl.kernel(
        _combine_body,
        mesh=mesh,
        compiler_params=pltpu.CompilerParams(needs_layout_passes=False),
        out_type=jax.ShapeDtypeStruct((T, D_MODEL), jnp.float32),
        scratch_types=[
            pltpu.VMEM((2, chunk), jnp.int32),
            pltpu.VMEM((chunk, D_MODEL), jnp.float32),
            pltpu.VMEM((chunk, D_MODEL), jnp.float32),
            pltpu.SemaphoreType.DMA,
            pltpu.SemaphoreType.DMA,
        ],
    )(out_rows, d0, d1)


def kernel(inputs, router_w, W1, W3, W2):
    batch, seq, dm = inputs.shape
    x = inputs.reshape(T, D_MODEL)
    dest, gates, te8 = _router(x, router_w)
    d0 = dest[:, 0]
    d1 = dest[:, 1]
    g0 = gates[:, 0]
    g1 = gates[:, 1]
    te = te8.reshape(E * E)[:NT]
    xs, rg = _dispatch(d0, d1, g0, g1, x)
    out_rows = _gemm(te, xs, W1, W3, W2, rg)
    final = _combine(out_rows, d0, d1)
    return final.reshape(batch, seq, dm)


# speedup vs baseline: 1.0054x; 1.0054x over previous
import functools

import jax
import jax.numpy as jnp
from jax import lax
from jax.experimental import pallas as pl
from jax.experimental.pallas import tpu as pltpu
from jax.experimental.pallas import tpu_sc as plsc

E = 8
TOP_K = 2
D_MODEL = 1024
D_FF = 4096
T = 2048
BM = 256
NT = 24
NTOT = NT * BM
BF = 1024
NF = D_FF // BF
EL = 128


def _router_body(x_ref, rw_ref, dest_ref, gates_ref, te_ref):
    i32 = jnp.int32
    x = x_ref[...]
    rw = rw_ref[...]
    logits = jnp.dot(x, rw, preferred_element_type=jnp.float32)
    eidx = lax.broadcasted_iota(i32, (T, EL), 1)
    logits = jnp.where(eidx < E, logits, -1e30)

    m1 = jnp.max(logits, axis=1, keepdims=True)
    i1 = jnp.min(jnp.where(logits == m1, eidx, EL), axis=1, keepdims=True)
    masked = jnp.where(eidx == i1, -1e30, logits)
    m2 = jnp.max(masked, axis=1, keepdims=True)
    i2 = jnp.min(jnp.where(masked == m2, eidx, EL), axis=1, keepdims=True)
    g1 = 1.0 / (1.0 + jnp.exp(m2 - m1))
    g2 = 1.0 - g1

    c1 = (eidx == i1).astype(i32)
    c2 = (eidx == i2).astype(i32)
    cnt = c1 + c2
    inc = cnt
    sh = 1
    while sh < T:
        inc = inc + jnp.concatenate(
            [jnp.zeros((sh, EL), i32), inc[:-sh]], axis=0)
        sh *= 2
    exc = inc - cnt
    counts = inc[T - 1:T, :]
    padded = ((counts + (BM - 1)) // BM) * BM
    start = jnp.zeros((1, EL), i32)
    for k in range(1, E):
        start = start + jnp.concatenate(
            [jnp.zeros((1, k), i32), padded[:, :-k]], axis=1)

    d1 = jnp.sum(c1 * (start + exc), axis=1, keepdims=True)
    d2 = jnp.sum(c2 * (start + exc), axis=1, keepdims=True)
    dest_ref[...] = jnp.concatenate([d1, d2], axis=1)
    gates_ref[...] = jnp.concatenate([g1, g2], axis=1)

    er = lax.broadcasted_iota(i32, (E, E), 0)
    ec = lax.broadcasted_iota(i32, (E, E), 1)
    tf = er * E + ec
    ec1 = lax.broadcasted_iota(i32, (1, EL), 1)
    te = jnp.full((E, E), -1, i32)
    for e in range(E):
        s_e = jnp.sum(jnp.where(ec1 == e, start, 0), axis=1, keepdims=True)
        te = te + (tf * BM >= s_e).astype(i32)
    te_ref[...] = jnp.clip(te, 0, E - 1)


def _router(x, router_w):
    rw = jnp.pad(router_w, ((0, 0), (0, EL - E)))
    return pl.pallas_call(
        _router_body,
        out_shape=[
            jax.ShapeDtypeStruct((T, 2), jnp.int32),
            jax.ShapeDtypeStruct((T, 2), jnp.float32),
            jax.ShapeDtypeStruct((E, E), jnp.int32),
        ],
    )(x, rw)


def _dispatch_body(d0_hbm, d1_hbm, g0_hbm, g1_hbm, x_hbm, xs_hbm, rg_hbm,
                   d_v, g_v, rt_v, rg_v, rows_v, gsems, wsems):
    c = lax.axis_index("c")
    s = lax.axis_index("s")
    w = c * 16 + s
    rows_per = NTOT // 32
    chunk = rows_per // 2
    lo = w * rows_per

    pltpu.sync_copy(d0_hbm, d_v.at[0])
    pltpu.sync_copy(d1_hbm, d_v.at[1])
    pltpu.sync_copy(g0_hbm, g_v.at[0])
    pltpu.sync_copy(g1_hbm, g_v.at[1])

    def zero(i, carry):
        rt_v[pl.ds(i * 16, 16)] = jnp.zeros((16,), jnp.int32)
        rg_v[pl.ds(i * 16, 16)] = jnp.zeros((16,), jnp.float32)
        return carry
    lax.fori_loop(0, rows_per // 16, zero, 0)

    for k in range(2):
        def scat(i, carry):
            sl = pl.ds(i * 16, 16)
            idx = d_v[k, sl]
            local = idx - lo
            m = (idx >= lo) & (local < rows_per)
            toks = lax.iota(jnp.int32, 16) + i * 16
            plsc.store_scatter(rt_v, [local], toks, mask=m)
            plsc.store_scatter(rg_v, [local], g_v[k, sl], mask=m)
            return carry
        lax.fori_loop(0, T // 16, scat, 0)

    pltpu.sync_copy(rg_v, rg_hbm.at[pl.ds(lo, rows_per)])

    nchunk = 6
    cr = rows_per // nchunk

    def gather(k):
        idx_slice = rt_v.at[pl.ds(k * cr, cr)]
        return pltpu.async_copy(x_hbm.at[idx_slice], rows_v.at[k % 3],
                                gsems.at[k % 3])

    def write(k):
        return pltpu.async_copy(rows_v.at[k % 3],
                                xs_hbm.at[pl.ds(lo + k * cr, cr)],
                                wsems.at[k % 3])

    gcp = {k: gather(k) for k in range(3)}
    wcp = {}
    for k in range(nchunk):
        gcp[k].wait()
        wcp[k] = write(k)
        if k + 3 < nchunk:
            wcp[k].wait()
            gcp[k + 3] = gather(k + 3)
    for k in range(nchunk - 3, nchunk):
        wcp[k].wait()


def _dispatch(d0, d1, g0, g1, x):
    mesh = plsc.VectorSubcoreMesh(core_axis_name="c", subcore_axis_name="s")
    rows_per = NTOT // 32
    chunk = rows_per // 2
    return pl.kernel(
        _dispatch_body,
        mesh=mesh,
        compiler_params=pltpu.CompilerParams(needs_layout_passes=False),
        out_type=[
            jax.ShapeDtypeStruct((NTOT, D_MODEL), jnp.float32),
            jax.ShapeDtypeStruct((NTOT,), jnp.float32),
        ],
        scratch_types=[
            pltpu.VMEM((2, T), jnp.int32),
            pltpu.VMEM((2, T), jnp.float32),
            pltpu.VMEM((rows_per,), jnp.int32),
            pltpu.VMEM((rows_per,), jnp.float32),
            pltpu.VMEM((3, rows_per // 6, D_MODEL), jnp.float32),
            pltpu.SemaphoreType.DMA((3,)),
            pltpu.SemaphoreType.DMA((3,)),
        ],
    )(d0, d1, g0, g1, x)


def _gemm_body(te_ref, acc_ref, xs_ref, w1_ref, w3_ref, w2_ref, rg_ref,
               out_ref):
    j = pl.program_id(0)
    x = xs_ref[...].astype(jnp.bfloat16)
    w1 = w1_ref[0].astype(jnp.bfloat16)
    w3 = w3_ref[0].astype(jnp.bfloat16)
    w2 = w2_ref[0].astype(jnp.bfloat16)
    a = jnp.dot(x, w1, preferred_element_type=jnp.float32)
    b = jnp.dot(x, w3, preferred_element_type=jnp.float32)
    h = (a / (1.0 + jnp.exp(-a))) * b
    contrib = jnp.dot(h.astype(jnp.bfloat16), w2,
                      preferred_element_type=jnp.float32)

    @pl.when(j == 0)
    def _():
        out_ref[...] = contrib

    @pl.when((j > 0) & (j < NF - 1))
    def _():
        out_ref[...] = acc_ref[...] + contrib

    @pl.when(j == NF - 1)
    def _():
        out_ref[...] = (acc_ref[...] + contrib) * rg_ref[0]


def _gemm(te, xs, W1, W3, W2, rg):
    grid_spec = pltpu.PrefetchScalarGridSpec(
        num_scalar_prefetch=1,
        grid=(NF, NT),
        in_specs=[
            pl.BlockSpec((BM, D_MODEL), lambda j, i, te: (i, 0)),
            pl.BlockSpec((BM, D_MODEL), lambda j, i, te: (i, 0)),
            pl.BlockSpec((1, D_MODEL, BF), lambda j, i, te: (te[i], 0, j)),
            pl.BlockSpec((1, D_MODEL, BF), lambda j, i, te: (te[i], 0, j)),
            pl.BlockSpec((1, BF, D_MODEL), lambda j, i, te: (te[i], j, 0)),
            pl.BlockSpec((1, BM, 1), lambda j, i, te: (i, 0, 0)),
        ],
        out_specs=pl.BlockSpec((BM, D_MODEL), lambda j, i, te: (i, 0)),
    )
    acc0 = jnp.zeros((NTOT, D_MODEL), jnp.float32)
    return pl.pallas_call(
        _gemm_body,
        grid_spec=grid_spec,
        out_shape=jax.ShapeDtypeStruct((NTOT, D_MODEL), jnp.float32),
        input_output_aliases={1: 0},
        compiler_params=pltpu.CompilerParams(
            dimension_semantics=("arbitrary", "arbitrary")),
    )(te, acc0, xs, W1, W3, W2, rg.reshape(NT, BM, 1))


def _combine_body(or_hbm, d0_hbm, d1_hbm, out_hbm,
                  idx_v, r0_v, r1_v, sem0, sem1):
    c = lax.axis_index("c")
    s = lax.axis_index("s")
    wid = c * 16 + s
    toks_per = T // 32
    chunk = toks_per // 2
    base = wid * toks_per
    for k in range(2):
        tb = base + k * chunk
        pltpu.sync_copy(d0_hbm.at[pl.ds(tb, chunk)], idx_v.at[0])
        pltpu.sync_copy(d1_hbm.at[pl.ds(tb, chunk)], idx_v.at[1])
        cp0 = pltpu.async_copy(or_hbm.at[idx_v.at[0]], r0_v, sem0)
        cp1 = pltpu.async_copy(or_hbm.at[idx_v.at[1]], r1_v, sem1)
        cp0.wait()
        cp1.wait()

        def row(r, carry):
            for cc in range(D_MODEL // 16):
                sl = pl.ds(cc * 16, 16)
                r0_v[r, sl] = r0_v[r, sl] + r1_v[r, sl]
            return carry
        lax.fori_loop(0, chunk, row, 0)
        pltpu.sync_copy(r0_v, out_hbm.at[pl.ds(tb, chunk)])


def _combine(out_rows, d0, d1):
    mesh = plsc.VectorSubcoreMesh(core_axis_name="c", subcore_axis_name="s")
    chunk = T // 64
    return pl.kernel(
        _combine_body,
        mesh=mesh,
        compiler_params=pltpu.CompilerParams(needs_layout_passes=False),
        out_type=jax.ShapeDtypeStruct((T, D_MODEL), jnp.float32),
        scratch_types=[
            pltpu.VMEM((2, chunk), jnp.int32),
            pltpu.VMEM((chunk, D_MODEL), jnp.float32),
            pltpu.VMEM((chunk, D_MODEL), jnp.float32),
            pltpu.SemaphoreType.DMA,
            pltpu.SemaphoreType.DMA,
        ],
    )(out_rows, d0, d1)


def kernel(inputs, router_w, W1, W3, W2):
    batch, seq, dm = inputs.shape
    x = inputs.reshape(T, D_MODEL)
    dest, gates, te8 = _router(x, router_w)
    d0 = dest[:, 0]
    d1 = dest[:, 1]
    g0 = gates[:, 0]
    g1 = gates[:, 1]
    te = te8.reshape(E * E)[:NT]
    xs, rg = _dispatch(d0, d1, g0, g1, x)
    out_rows = _gemm(te, xs, W1, W3, W2, rg)
    final = _combine(out_rows, d0, d1)
    return final.reshape(batch, seq, dm)


# speedup vs baseline: 1.1030x; 1.0971x over previous
import functools

import jax
import jax.numpy as jnp
from jax import lax
from jax.experimental import pallas as pl
from jax.experimental.pallas import tpu as pltpu
from jax.experimental.pallas import tpu_sc as plsc

E = 8
TOP_K = 2
D_MODEL = 1024
D_FF = 4096
T = 2048
BM = 256
NT = 24
NTOT = NT * BM
BF = 2048
NF = D_FF // BF
EL = 128


def _router_body(x_ref, rw_ref, dest_ref, gates_ref, te_ref):
    i32 = jnp.int32
    x = x_ref[...]
    rw = rw_ref[...]
    logits = jnp.dot(x, rw, preferred_element_type=jnp.float32)
    eidx = lax.broadcasted_iota(i32, (T, EL), 1)
    logits = jnp.where(eidx < E, logits, -1e30)

    m1 = jnp.max(logits, axis=1, keepdims=True)
    i1 = jnp.min(jnp.where(logits == m1, eidx, EL), axis=1, keepdims=True)
    masked = jnp.where(eidx == i1, -1e30, logits)
    m2 = jnp.max(masked, axis=1, keepdims=True)
    i2 = jnp.min(jnp.where(masked == m2, eidx, EL), axis=1, keepdims=True)
    g1 = 1.0 / (1.0 + jnp.exp(m2 - m1))
    g2 = 1.0 - g1

    c1 = (eidx == i1).astype(i32)
    c2 = (eidx == i2).astype(i32)
    cnt = c1 + c2
    inc = cnt
    sh = 1
    while sh < T:
        inc = inc + jnp.concatenate(
            [jnp.zeros((sh, EL), i32), inc[:-sh]], axis=0)
        sh *= 2
    exc = inc - cnt
    counts = inc[T - 1:T, :]
    padded = ((counts + (BM - 1)) // BM) * BM
    start = jnp.zeros((1, EL), i32)
    for k in range(1, E):
        start = start + jnp.concatenate(
            [jnp.zeros((1, k), i32), padded[:, :-k]], axis=1)

    d1 = jnp.sum(c1 * (start + exc), axis=1, keepdims=True)
    d2 = jnp.sum(c2 * (start + exc), axis=1, keepdims=True)
    dest_ref[...] = jnp.concatenate([d1, d2], axis=1)
    gates_ref[...] = jnp.concatenate([g1, g2], axis=1)

    er = lax.broadcasted_iota(i32, (E, E), 0)
    ec = lax.broadcasted_iota(i32, (E, E), 1)
    tf = er * E + ec
    ec1 = lax.broadcasted_iota(i32, (1, EL), 1)
    te = jnp.full((E, E), -1, i32)
    for e in range(E):
        s_e = jnp.sum(jnp.where(ec1 == e, start, 0), axis=1, keepdims=True)
        te = te + (tf * BM >= s_e).astype(i32)
    te_ref[...] = jnp.clip(te, 0, E - 1)


def _router(x, router_w):
    rw = jnp.pad(router_w, ((0, 0), (0, EL - E)))
    return pl.pallas_call(
        _router_body,
        out_shape=[
            jax.ShapeDtypeStruct((T, 2), jnp.int32),
            jax.ShapeDtypeStruct((T, 2), jnp.float32),
            jax.ShapeDtypeStruct((E, E), jnp.int32),
        ],
    )(x, rw)


def _dispatch_body(d0_hbm, d1_hbm, g0_hbm, g1_hbm, x_hbm, xs_hbm, rg_hbm,
                   d_v, g_v, rt_v, rg_v, rows_v, gsems, wsems):
    c = lax.axis_index("c")
    s = lax.axis_index("s")
    w = c * 16 + s
    rows_per = NTOT // 32
    chunk = rows_per // 2
    lo = w * rows_per

    pltpu.sync_copy(d0_hbm, d_v.at[0])
    pltpu.sync_copy(d1_hbm, d_v.at[1])
    pltpu.sync_copy(g0_hbm, g_v.at[0])
    pltpu.sync_copy(g1_hbm, g_v.at[1])

    def zero(i, carry):
        rt_v[i >> 1, pl.ds((i & 1) * 16, 16)] = jnp.zeros((16,), jnp.int32)
        rg_v[pl.ds(i * 16, 16)] = jnp.zeros((16,), jnp.float32)
        return carry
    lax.fori_loop(0, rows_per // 16, zero, 0)

    for k in range(2):
        def scat(i, carry):
            sl = pl.ds(i * 16, 16)
            idx = d_v[k, sl]
            local = idx - lo
            m = (idx >= lo) & (local < rows_per)
            toks = lax.iota(jnp.int32, 16) + i * 16
            plsc.store_scatter(rt_v, [local >> 5, local & 31], toks, mask=m)
            plsc.store_scatter(rg_v, [local], g_v[k, sl], mask=m)
            return carry
        lax.fori_loop(0, T // 16, scat, 0)

    pltpu.sync_copy(rg_v, rg_hbm.at[pl.ds(lo, rows_per)])

    nchunk = 6
    cr = rows_per // nchunk

    def gather(k):
        return pltpu.async_copy(x_hbm.at[rt_v.at[k]], rows_v.at[k % 3],
                                gsems.at[k % 3])

    def write(k):
        return pltpu.async_copy(rows_v.at[k % 3],
                                xs_hbm.at[pl.ds(lo + k * cr, cr)],
                                wsems.at[k % 3])

    gcp = {k: gather(k) for k in range(3)}
    wcp = {}
    for k in range(nchunk):
        gcp[k].wait()
        wcp[k] = write(k)
        if k + 3 < nchunk:
            wcp[k].wait()
            gcp[k + 3] = gather(k + 3)
    for k in range(nchunk - 3, nchunk):
        wcp[k].wait()


def _dispatch(d0, d1, g0, g1, x):
    mesh = plsc.VectorSubcoreMesh(core_axis_name="c", subcore_axis_name="s")
    rows_per = NTOT // 32
    chunk = rows_per // 2
    return pl.kernel(
        _dispatch_body,
        mesh=mesh,
        compiler_params=pltpu.CompilerParams(needs_layout_passes=False),
        out_type=[
            jax.ShapeDtypeStruct((NTOT, D_MODEL), jnp.float32),
            jax.ShapeDtypeStruct((NTOT,), jnp.float32),
        ],
        scratch_types=[
            pltpu.VMEM((2, T), jnp.int32),
            pltpu.VMEM((2, T), jnp.float32),
            pltpu.VMEM((6, rows_per // 6), jnp.int32),
            pltpu.VMEM((rows_per,), jnp.float32),
            pltpu.VMEM((3, rows_per // 6, D_MODEL), jnp.float32),
            pltpu.SemaphoreType.DMA((3,)),
            pltpu.SemaphoreType.DMA((3,)),
        ],
    )(d0, d1, g0, g1, x)


def _gemm_body(te_ref, acc_ref, xs_ref, w1_ref, w3_ref, w2_ref, rg_ref,
               out_ref):
    j = pl.program_id(0)
    x = xs_ref[...].astype(jnp.bfloat16)
    w1 = w1_ref[0].astype(jnp.bfloat16)
    w3 = w3_ref[0].astype(jnp.bfloat16)
    w2 = w2_ref[0].astype(jnp.bfloat16)
    a = jnp.dot(x, w1, preferred_element_type=jnp.float32)
    b = jnp.dot(x, w3, preferred_element_type=jnp.float32)
    h = (a / (1.0 + jnp.exp(-a))) * b
    contrib = jnp.dot(h.astype(jnp.bfloat16), w2,
                      preferred_element_type=jnp.float32)

    @pl.when(j == 0)
    def _():
        out_ref[...] = contrib

    @pl.when((j > 0) & (j < NF - 1))
    def _():
        out_ref[...] = acc_ref[...] + contrib

    @pl.when(j == NF - 1)
    def _():
        out_ref[...] = (acc_ref[...] + contrib) * rg_ref[0]


def _gemm(te, xs, W1, W3, W2, rg):
    grid_spec = pltpu.PrefetchScalarGridSpec(
        num_scalar_prefetch=1,
        grid=(NF, NT),
        in_specs=[
            pl.BlockSpec((BM, D_MODEL), lambda j, i, te: (i, 0)),
            pl.BlockSpec((BM, D_MODEL), lambda j, i, te: (i, 0)),
            pl.BlockSpec((1, D_MODEL, BF), lambda j, i, te: (te[i], 0, j)),
            pl.BlockSpec((1, D_MODEL, BF), lambda j, i, te: (te[i], 0, j)),
            pl.BlockSpec((1, BF, D_MODEL), lambda j, i, te: (te[i], j, 0)),
            pl.BlockSpec((1, BM, 1), lambda j, i, te: (i, 0, 0)),
        ],
        out_specs=pl.BlockSpec((BM, D_MODEL), lambda j, i, te: (i, 0)),
    )
    acc0 = jnp.zeros((NTOT, D_MODEL), jnp.float32)
    return pl.pallas_call(
        _gemm_body,
        grid_spec=grid_spec,
        out_shape=jax.ShapeDtypeStruct((NTOT, D_MODEL), jnp.float32),
        input_output_aliases={1: 0},
        compiler_params=pltpu.CompilerParams(
            dimension_semantics=("arbitrary", "arbitrary")),
    )(te, acc0, xs, W1, W3, W2, rg.reshape(NT, BM, 1))


def _combine_body(or_hbm, d0_hbm, d1_hbm, out_hbm,
                  idx_v, r0_v, r1_v, sem0, sem1):
    c = lax.axis_index("c")
    s = lax.axis_index("s")
    wid = c * 16 + s
    toks_per = T // 32
    chunk = toks_per // 2
    base = wid * toks_per
    for k in range(2):
        tb = base + k * chunk
        pltpu.sync_copy(d0_hbm.at[pl.ds(tb, chunk)], idx_v.at[0])
        pltpu.sync_copy(d1_hbm.at[pl.ds(tb, chunk)], idx_v.at[1])
        cp0 = pltpu.async_copy(or_hbm.at[idx_v.at[0]], r0_v, sem0)
        cp1 = pltpu.async_copy(or_hbm.at[idx_v.at[1]], r1_v, sem1)
        cp0.wait()
        cp1.wait()

        def row(r, carry):
            for cc in range(D_MODEL // 16):
                sl = pl.ds(cc * 16, 16)
                r0_v[r, sl] = r0_v[r, sl] + r1_v[r, sl]
            return carry
        lax.fori_loop(0, chunk, row, 0)
        pltpu.sync_copy(r0_v, out_hbm.at[pl.ds(tb, chunk)])


def _combine(out_rows, d0, d1):
    mesh = plsc.VectorSubcoreMesh(core_axis_name="c", subcore_axis_name="s")
    chunk = T // 64
    return pl.kernel(
        _combine_body,
        mesh=mesh,
        compiler_params=pltpu.CompilerParams(needs_layout_passes=False),
        out_type=jax.ShapeDtypeStruct((T, D_MODEL), jnp.float32),
        scratch_types=[
            pltpu.VMEM((2, chunk), jnp.int32),
            pltpu.VMEM((chunk, D_MODEL), jnp.float32),
            pltpu.VMEM((chunk, D_MODEL), jnp.float32),
            pltpu.SemaphoreType.DMA,
            pltpu.SemaphoreType.DMA,
        ],
    )(out_rows, d0, d1)


def kernel(inputs, router_w, W1, W3, W2):
    batch, seq, dm = inputs.shape
    x = inputs.reshape(T, D_MODEL)
    dest, gates, te8 = _router(x, router_w)
    d0 = dest[:, 0]
    d1 = dest[:, 1]
    g0 = gates[:, 0]
    g1 = gates[:, 1]
    te = te8.reshape(E * E)[:NT]
    xs, rg = _dispatch(d0, d1, g0, g1, x)
    out_rows = _gemm(te, xs, W1, W3, W2, rg)
    final = _combine(out_rows, d0, d1)
    return final.reshape(batch, seq, dm)
